# Initial kernel scaffold; baseline (speedup 1.0000x reference)
import jax
import jax.numpy as jnp
from jax.experimental import pallas as pl


def kernel(features_1, features_2, edges_1, edges_2, W1, b1, W2, b2, W3, b3, att_W, ntn_W, ntn_V, ntn_b, fc_W, fc_b, out_W, out_b):
    raise NotImplementedError("write your pallas kernel here")


# speedup vs baseline: 5.5669x; 5.5669x over previous
import functools

import jax
import jax.numpy as jnp
from jax import lax
from jax.experimental import pallas as pl
from jax.experimental.pallas import tpu as pltpu
from jax.experimental.pallas import tpu_sc as plsc

N = 10000
E = 320000
NS = 16
NPAD = 10240
RPT = NPAD // NS
CHUNK = 128
EPT = 160 * CHUNK
EPAD = NS * EPT
NCHUNK = EPT // CHUNK
DEGW = 8

_MESH = plsc.VectorSubcoreMesh(core_axis_name="c", subcore_axis_name="s")


def _deg_body(d1, d2, ones_hbm, zeros_hbm, deg1, deg2, idx_v, ones_v, acc):
    c = lax.axis_index("c")
    s = lax.axis_index("s")

    def work(d_ref, deg_ref):
        pltpu.sync_copy(ones_hbm, ones_v)
        pltpu.sync_copy(zeros_hbm.at[pl.ds(s * RPT, RPT)],
                        acc.at[pl.ds(s * RPT, RPT)])
        plsc.subcore_barrier()

        def chunk(j, carry):
            base = s * EPT + j * CHUNK
            pltpu.sync_copy(d_ref.at[pl.ds(base, CHUNK)], idx_v)
            pltpu.sync_copy(ones_v, acc.at[idx_v], add=True)
            return carry

        lax.fori_loop(0, NCHUNK, chunk, 0)
        plsc.subcore_barrier()
        pltpu.sync_copy(acc.at[pl.ds(s * RPT, RPT)],
                        deg_ref.at[pl.ds(s * RPT, RPT)])

    @pl.when(c == 0)
    def _():
        work(d1, deg1)

    @pl.when(c == 1)
    def _():
        work(d2, deg2)


def _sc_degree(d1, d2, ones, zeros):
    f = pl.kernel(
        _deg_body,
        out_type=(jax.ShapeDtypeStruct((NPAD, DEGW), jnp.float32),
                  jax.ShapeDtypeStruct((NPAD, DEGW), jnp.float32)),
        mesh=_MESH,
        scratch_types=[
            pltpu.VMEM((CHUNK,), jnp.int32),
            pltpu.VMEM((CHUNK, DEGW), jnp.float32),
            pltpu.VMEM_SHARED((NPAD, DEGW), jnp.float32),
        ],
    )
    return f(d1, d2, ones, zeros)


def _agg_body(xws1, xws2, s1, d1, s2, d2, out1, out2,
              sidx_v, didx_v, rows_v, acc, sem):
    c = lax.axis_index("c")
    s = lax.axis_index("s")

    def work(xws_ref, s_ref, d_ref, out_ref):
        pltpu.sync_copy(xws_ref.at[pl.ds(s * RPT, RPT)],
                        acc.at[pl.ds(s * RPT, RPT)])
        plsc.subcore_barrier()

        def chunk(j, carry):
            base = s * EPT + j * CHUNK
            pltpu.sync_copy(s_ref.at[pl.ds(base, CHUNK)], sidx_v)
            pltpu.sync_copy(d_ref.at[pl.ds(base, CHUNK)], didx_v)
            pltpu.async_copy(xws_ref.at[sidx_v], rows_v, sem).wait()
            pltpu.sync_copy(rows_v, acc.at[didx_v], add=True)
            return carry

        lax.fori_loop(0, NCHUNK, chunk, 0)
        plsc.subcore_barrier()
        pltpu.sync_copy(acc.at[pl.ds(s * RPT, RPT)],
                        out_ref.at[pl.ds(s * RPT, RPT)])

    @pl.when(c == 0)
    def _():
        work(xws1, s1, d1, out1)

    @pl.when(c == 1)
    def _():
        work(xws2, s2, d2, out2)


@functools.cache
def _sc_agg(C):
    return pl.kernel(
        _agg_body,
        out_type=(jax.ShapeDtypeStruct((NPAD, C), jnp.float32),
                  jax.ShapeDtypeStruct((NPAD, C), jnp.float32)),
        mesh=_MESH,
        scratch_types=[
            pltpu.VMEM((CHUNK,), jnp.int32),
            pltpu.VMEM((CHUNK,), jnp.int32),
            pltpu.VMEM((CHUNK, C), jnp.float32),
            pltpu.VMEM_SHARED((NPAD, C), jnp.float32),
            pltpu.SemaphoreType.DMA,
        ],
    )


BR = 1024


def _mm1_body(x_ref, w_ref, deg_ref, xws_ref, dinv_ref):
    dinv = lax.rsqrt(deg_ref[...] + 1.0)
    xw = jnp.dot(x_ref[...], w_ref[...], preferred_element_type=jnp.float32)
    xws_ref[...] = xw * dinv
    dinv_ref[...] = dinv


def _tc_layer1(x, w, degp):
    cin, cout = w.shape
    return pl.pallas_call(
        _mm1_body,
        grid=(NPAD // BR,),
        in_specs=[
            pl.BlockSpec((BR, cin), lambda i: (i, 0)),
            pl.BlockSpec((cin, cout), lambda i: (0, 0)),
            pl.BlockSpec((BR, 1), lambda i: (i, 0)),
        ],
        out_specs=[
            pl.BlockSpec((BR, cout), lambda i: (i, 0)),
            pl.BlockSpec((BR, 1), lambda i: (i, 0)),
        ],
        out_shape=[
            jax.ShapeDtypeStruct((NPAD, cout), jnp.float32),
            jax.ShapeDtypeStruct((NPAD, 1), jnp.float32),
        ],
    )(x, w, degp)


def _mid_body(s_ref, w_ref, dinv_ref, b_ref, out_ref):
    dinv = dinv_ref[...]
    h = jnp.maximum(s_ref[...] * dinv + b_ref[...], 0.0)
    out_ref[...] = jnp.dot(h, w_ref[...],
                           preferred_element_type=jnp.float32) * dinv


def _tc_mid(sagg, w, dinv, b_row):
    cin, cout = w.shape
    return pl.pallas_call(
        _mid_body,
        grid=(NPAD // BR,),
        in_specs=[
            pl.BlockSpec((BR, cin), lambda i: (i, 0)),
            pl.BlockSpec((cin, cout), lambda i: (0, 0)),
            pl.BlockSpec((BR, 1), lambda i: (i, 0)),
            pl.BlockSpec((1, cin), lambda i: (0, 0)),
        ],
        out_specs=pl.BlockSpec((BR, cout), lambda i: (i, 0)),
        out_shape=jax.ShapeDtypeStruct((NPAD, cout), jnp.float32),
    )(sagg, w, dinv, b_row)


def _tail_body(s31, s32, dinv1, dinv2, b3r, attw, w2, vt, ntnbT,
               fcw, fcbr, outw, outbr, out_ref):
    def pool(s_ref, dinv_ref):
        h3 = s_ref[...] * dinv_ref[...] + b3r[...]
        ga = jnp.dot(h3, attw[...], preferred_element_type=jnp.float32)
        gc = jnp.mean(ga, axis=0, keepdims=True)
        tg = jnp.tanh(gc)
        sig = jax.nn.sigmoid(jnp.sum(h3 * tg, axis=1, keepdims=True))
        return jnp.sum(h3 * sig, axis=0, keepdims=True)

    g1 = pool(s31, dinv1)
    g2 = pool(s32, dinv2)
    tmp = jnp.dot(g1, w2[...], preferred_element_type=jnp.float32)
    parts = [
        jnp.sum(tmp[:, t * 32:(t + 1) * 32] * g2, axis=1, keepdims=True)
        for t in range(16)
    ]
    scoring = jnp.concatenate(parts, axis=1)
    comb = jnp.concatenate([g1, g2], axis=1)
    block = jnp.dot(comb, vt[...], preferred_element_type=jnp.float32)
    scores = jnp.maximum(scoring + block + ntnbT[...], 0.0)
    h = jnp.maximum(
        jnp.dot(scores, fcw[...], preferred_element_type=jnp.float32)
        + fcbr[...], 0.0)
    out_ref[...] = jax.nn.sigmoid(
        jnp.dot(h, outw[...], preferred_element_type=jnp.float32)
        + outbr[...])


def _tc_tail(s31, s32, dinv1, dinv2, b3r, attw, w2, vt, ntnbT,
             fcw, fcbr, outw, outbr):
    return pl.pallas_call(
        _tail_body,
        out_shape=jax.ShapeDtypeStruct((1, 1), jnp.float32),
    )(s31, s32, dinv1, dinv2, b3r, attw, w2, vt, ntnbT,
      fcw, fcbr, outw, outbr)


def kernel(features_1, features_2, edges_1, edges_2, W1, b1, W2, b2, W3, b3,
           att_W, ntn_W, ntn_V, ntn_b, fc_W, fc_b, out_W, out_b):
    e1 = edges_1.astype(jnp.int32)
    e2 = edges_2.astype(jnp.int32)
    pad = jnp.full((EPAD - E,), NPAD - 1, jnp.int32)
    src1 = jnp.concatenate([e1[0], pad])
    dst1 = jnp.concatenate([e1[1], pad])
    src2 = jnp.concatenate([e2[0], pad])
    dst2 = jnp.concatenate([e2[1], pad])

    deg1 = jnp.zeros((NPAD,), jnp.float32).at[dst1].add(1.0)[:, None]
    deg2 = jnp.zeros((NPAD,), jnp.float32).at[dst2].add(1.0)[:, None]

    x1 = jnp.pad(features_1, ((0, NPAD - N), (0, 0)))
    x2 = jnp.pad(features_2, ((0, NPAD - N), (0, 0)))
    xws1, dinv1 = _tc_layer1(x1, W1, deg1)
    xws2, dinv2 = _tc_layer1(x2, W1, deg2)

    W2p = jnp.pad(W2, ((0, 0), (0, 128 - W2.shape[1])))
    W3p = jnp.pad(W3, ((0, 128 - W3.shape[0]), (0, 128 - W3.shape[1])))
    b2p = jnp.pad(b2, (0, 128 - b2.shape[0]))

    s11, s12 = _sc_agg(128)(xws1, xws2, src1, dst1, src2, dst2)
    b1r = b1.reshape(1, -1)
    xws1 = _tc_mid(s11, W2p, dinv1, b1r)
    xws2 = _tc_mid(s12, W2p, dinv2, b1r)

    s21, s22 = _sc_agg(128)(xws1, xws2, src1, dst1, src2, dst2)
    b2r = b2p.reshape(1, -1)
    xws1 = _tc_mid(s21, W3p, dinv1, b2r)
    xws2 = _tc_mid(s22, W3p, dinv2, b2r)

    s31, s32 = _sc_agg(128)(xws1, xws2, src1, dst1, src2, dst2)

    w2t = ntn_W.transpose(0, 2, 1).reshape(32, 512)
    return _tc_tail(
        s31[:N, :32], s32[:N, :32], dinv1[:N], dinv2[:N],
        b3.reshape(1, -1), att_W, w2t, ntn_V.T, ntn_b.reshape(1, -1),
        fc_W, fc_b.reshape(1, -1), out_W, out_b.reshape(1, -1))


# speedup vs baseline: 5.9529x; 1.0693x over previous
import functools

import jax
import jax.numpy as jnp
from jax import lax
from jax.experimental import pallas as pl
from jax.experimental.pallas import tpu as pltpu
from jax.experimental.pallas import tpu_sc as plsc

N = 10000
E = 320000
NS = 16
NPAD = 10240
RPT = NPAD // NS
CHUNK = 128
EPT = 160 * CHUNK
EPAD = NS * EPT
NCHUNK = EPT // CHUNK
DEGW = 8

_MESH = plsc.VectorSubcoreMesh(core_axis_name="c", subcore_axis_name="s")


def _deg_body(d1, d2, zeros8, ones8, deg1, deg2, didx_v, ones_v, acc):
    c = lax.axis_index("c")
    s = lax.axis_index("s")

    def work(d_ref, deg_ref):
        pltpu.sync_copy(zeros8.at[pl.ds(s * RPT, RPT)],
                        acc.at[pl.ds(s * RPT, RPT)])
        pltpu.sync_copy(ones8, ones_v)
        plsc.subcore_barrier()

        def chunk(j, carry):
            base = s * EPT + j * CHUNK
            pltpu.sync_copy(d_ref.at[pl.ds(base, CHUNK)], didx_v)
            pltpu.sync_copy(ones_v, acc.at[didx_v], add=True)
            return carry

        lax.fori_loop(0, NCHUNK, chunk, 0)
        plsc.subcore_barrier()
        pltpu.sync_copy(acc.at[pl.ds(s * RPT, RPT)],
                        deg_ref.at[pl.ds(s * RPT, RPT)])

    @pl.when(c == 0)
    def _():
        work(d1, deg1)

    @pl.when(c == 1)
    def _():
        work(d2, deg2)


def _sc_degree(d1, d2, zeros8, ones8):
    f = pl.kernel(
        _deg_body,
        out_type=(jax.ShapeDtypeStruct((NPAD, DEGW), jnp.float32),
                  jax.ShapeDtypeStruct((NPAD, DEGW), jnp.float32)),
        mesh=_MESH,
        scratch_types=[
            pltpu.VMEM((CHUNK,), jnp.int32),
            pltpu.VMEM((CHUNK, DEGW), jnp.float32),
            pltpu.VMEM_SHARED((NPAD, DEGW), jnp.float32),
        ],
    )
    return f(d1, d2, zeros8, ones8)


def _agg_body(xws1, xws2, s1, d1, s2, d2, out1, out2,
              sidx_v, didx_v, rows_v, acc, sem):
    c = lax.axis_index("c")
    s = lax.axis_index("s")

    def work(xws_ref, s_ref, d_ref, out_ref):
        pltpu.sync_copy(xws_ref.at[pl.ds(s * RPT, RPT)],
                        acc.at[pl.ds(s * RPT, RPT)])
        plsc.subcore_barrier()

        def chunk(j, carry):
            base = s * EPT + j * CHUNK
            pltpu.sync_copy(s_ref.at[pl.ds(base, CHUNK)], sidx_v)
            pltpu.sync_copy(d_ref.at[pl.ds(base, CHUNK)], didx_v)
            pltpu.async_copy(xws_ref.at[sidx_v], rows_v, sem).wait()
            pltpu.sync_copy(rows_v, acc.at[didx_v], add=True)
            return carry

        lax.fori_loop(0, NCHUNK, chunk, 0)
        plsc.subcore_barrier()
        pltpu.sync_copy(acc.at[pl.ds(s * RPT, RPT)],
                        out_ref.at[pl.ds(s * RPT, RPT)])

    @pl.when(c == 0)
    def _():
        work(xws1, s1, d1, out1)

    @pl.when(c == 1)
    def _():
        work(xws2, s2, d2, out2)


@functools.cache
def _sc_agg(C):
    return pl.kernel(
        _agg_body,
        out_type=(jax.ShapeDtypeStruct((NPAD, C), jnp.float32),
                  jax.ShapeDtypeStruct((NPAD, C), jnp.float32)),
        mesh=_MESH,
        scratch_types=[
            pltpu.VMEM((CHUNK,), jnp.int32),
            pltpu.VMEM((CHUNK,), jnp.int32),
            pltpu.VMEM((CHUNK, C), jnp.float32),
            pltpu.VMEM_SHARED((NPAD, C), jnp.float32),
            pltpu.SemaphoreType.DMA,
        ],
    )


BR = 1024


def _mm1_body(x_ref, w_ref, deg_ref, xws_ref, dinv_ref):
    dinv = lax.rsqrt(deg_ref[...] + 1.0)
    xw = jnp.dot(x_ref[...], w_ref[...], preferred_element_type=jnp.float32)
    xws_ref[...] = xw * dinv
    dinv_ref[...] = dinv


def _tc_layer1(x, w, degp):
    cin, cout = w.shape
    return pl.pallas_call(
        _mm1_body,
        grid=(NPAD // BR,),
        in_specs=[
            pl.BlockSpec((BR, cin), lambda i: (i, 0)),
            pl.BlockSpec((cin, cout), lambda i: (0, 0)),
            pl.BlockSpec((BR, 1), lambda i: (i, 0)),
        ],
        out_specs=[
            pl.BlockSpec((BR, cout), lambda i: (i, 0)),
            pl.BlockSpec((BR, 1), lambda i: (i, 0)),
        ],
        out_shape=[
            jax.ShapeDtypeStruct((NPAD, cout), jnp.float32),
            jax.ShapeDtypeStruct((NPAD, 1), jnp.float32),
        ],
    )(x, w, degp)


def _mid_body(s_ref, w_ref, dinv_ref, b_ref, out_ref):
    dinv = dinv_ref[...]
    h = jnp.maximum(s_ref[...] * dinv + b_ref[...], 0.0)
    out_ref[...] = jnp.dot(h, w_ref[...],
                           preferred_element_type=jnp.float32) * dinv


def _tc_mid(sagg, w, dinv, b_row):
    cin, cout = w.shape
    return pl.pallas_call(
        _mid_body,
        grid=(NPAD // BR,),
        in_specs=[
            pl.BlockSpec((BR, cin), lambda i: (i, 0)),
            pl.BlockSpec((cin, cout), lambda i: (0, 0)),
            pl.BlockSpec((BR, 1), lambda i: (i, 0)),
            pl.BlockSpec((1, cin), lambda i: (0, 0)),
        ],
        out_specs=pl.BlockSpec((BR, cout), lambda i: (i, 0)),
        out_shape=jax.ShapeDtypeStruct((NPAD, cout), jnp.float32),
    )(sagg, w, dinv, b_row)


def _tail_body(s31, s32, dinv1, dinv2, b3r, attw, w2, vt, ntnbT,
               fcw, fcbr, outw, outbr, out_ref):
    def pool(s_ref, dinv_ref):
        h3 = s_ref[...] * dinv_ref[...] + b3r[...]
        ga = jnp.dot(h3, attw[...], preferred_element_type=jnp.float32)
        gc = jnp.mean(ga, axis=0, keepdims=True)
        tg = jnp.tanh(gc)
        sig = jax.nn.sigmoid(jnp.sum(h3 * tg, axis=1, keepdims=True))
        return jnp.sum(h3 * sig, axis=0, keepdims=True)

    g1 = pool(s31, dinv1)
    g2 = pool(s32, dinv2)
    tmp = jnp.dot(g1, w2[...], preferred_element_type=jnp.float32)
    parts = [
        jnp.sum(tmp[:, t * 32:(t + 1) * 32] * g2, axis=1, keepdims=True)
        for t in range(16)
    ]
    scoring = jnp.concatenate(parts, axis=1)
    comb = jnp.concatenate([g1, g2], axis=1)
    block = jnp.dot(comb, vt[...], preferred_element_type=jnp.float32)
    scores = jnp.maximum(scoring + block + ntnbT[...], 0.0)
    h = jnp.maximum(
        jnp.dot(scores, fcw[...], preferred_element_type=jnp.float32)
        + fcbr[...], 0.0)
    out_ref[...] = jax.nn.sigmoid(
        jnp.dot(h, outw[...], preferred_element_type=jnp.float32)
        + outbr[...])


def _tc_tail(s31, s32, dinv1, dinv2, b3r, attw, w2, vt, ntnbT,
             fcw, fcbr, outw, outbr):
    return pl.pallas_call(
        _tail_body,
        out_shape=jax.ShapeDtypeStruct((1, 1), jnp.float32),
    )(s31, s32, dinv1, dinv2, b3r, attw, w2, vt, ntnbT,
      fcw, fcbr, outw, outbr)


def kernel(features_1, features_2, edges_1, edges_2, W1, b1, W2, b2, W3, b3,
           att_W, ntn_W, ntn_V, ntn_b, fc_W, fc_b, out_W, out_b):
    e1 = edges_1.astype(jnp.int32)
    e2 = edges_2.astype(jnp.int32)
    pad = jnp.full((EPAD - E,), NPAD - 1, jnp.int32)
    src1 = jnp.concatenate([e1[0], pad])
    dst1 = jnp.concatenate([e1[1], pad])
    src2 = jnp.concatenate([e2[0], pad])
    dst2 = jnp.concatenate([e2[1], pad])

    zeros8 = jnp.zeros((NPAD, DEGW), jnp.float32)
    ones8 = jnp.ones((CHUNK, DEGW), jnp.float32)
    deg1, deg2 = _sc_degree(dst1, dst2, zeros8, ones8)
    deg1 = deg1[:, :1]
    deg2 = deg2[:, :1]

    x1 = jnp.pad(features_1, ((0, NPAD - N), (0, 0)))
    x2 = jnp.pad(features_2, ((0, NPAD - N), (0, 0)))
    xws1, dinv1 = _tc_layer1(x1, W1, deg1)
    xws2, dinv2 = _tc_layer1(x2, W1, deg2)

    W2p = jnp.pad(W2, ((0, 0), (0, 128 - W2.shape[1])))
    W3p = jnp.pad(W3, ((0, 128 - W3.shape[0]), (0, 128 - W3.shape[1])))
    b2p = jnp.pad(b2, (0, 128 - b2.shape[0]))

    s11, s12 = _sc_agg(128)(xws1, xws2, src1, dst1, src2, dst2)
    b1r = b1.reshape(1, -1)
    xws1 = _tc_mid(s11, W2p, dinv1, b1r)
    xws2 = _tc_mid(s12, W2p, dinv2, b1r)

    s21, s22 = _sc_agg(128)(xws1, xws2, src1, dst1, src2, dst2)
    b2r = b2p.reshape(1, -1)
    xws1 = _tc_mid(s21, W3p, dinv1, b2r)
    xws2 = _tc_mid(s22, W3p, dinv2, b2r)

    s31, s32 = _sc_agg(128)(xws1, xws2, src1, dst1, src2, dst2)

    w2t = ntn_W.transpose(0, 2, 1).reshape(32, 512)
    return _tc_tail(
        s31[:N, :32], s32[:N, :32], dinv1[:N], dinv2[:N],
        b3.reshape(1, -1), att_W, w2t, ntn_V.T, ntn_b.reshape(1, -1),
        fc_W, fc_b.reshape(1, -1), out_W, out_b.reshape(1, -1))


# speedup vs baseline: 7.5150x; 1.2624x over previous
import functools

import jax
import jax.numpy as jnp
from jax import lax
from jax.experimental import pallas as pl
from jax.experimental.pallas import tpu as pltpu
from jax.experimental.pallas import tpu_sc as plsc

N = 10000
E = 320000
NS = 16
NPAD = 10240
RPT = NPAD // NS
CHUNK = 128
EPT = 160 * CHUNK
EPAD = NS * EPT
NCHUNK = EPT // CHUNK
DEGW = 8

_MESH = plsc.VectorSubcoreMesh(core_axis_name="c", subcore_axis_name="s")


def _deg_body(d1, d2, zeros8, ones8, deg1, deg2, didx_v, ones_v, acc):
    c = lax.axis_index("c")
    s = lax.axis_index("s")

    def work(d_ref, deg_ref):
        pltpu.sync_copy(zeros8.at[pl.ds(s * RPT, RPT)],
                        acc.at[pl.ds(s * RPT, RPT)])
        pltpu.sync_copy(ones8, ones_v)
        plsc.subcore_barrier()

        def chunk(j, carry):
            base = s * EPT + j * CHUNK
            pltpu.sync_copy(d_ref.at[pl.ds(base, CHUNK)], didx_v)
            pltpu.sync_copy(ones_v, acc.at[didx_v], add=True)
            return carry

        lax.fori_loop(0, NCHUNK, chunk, 0)
        plsc.subcore_barrier()
        pltpu.sync_copy(acc.at[pl.ds(s * RPT, RPT)],
                        deg_ref.at[pl.ds(s * RPT, RPT)])

    @pl.when(c == 0)
    def _():
        work(d1, deg1)

    @pl.when(c == 1)
    def _():
        work(d2, deg2)


def _sc_degree(d1, d2, zeros8, ones8):
    f = pl.kernel(
        _deg_body,
        out_type=(jax.ShapeDtypeStruct((NPAD, DEGW), jnp.float32),
                  jax.ShapeDtypeStruct((NPAD, DEGW), jnp.float32)),
        mesh=_MESH,
        scratch_types=[
            pltpu.VMEM((CHUNK,), jnp.int32),
            pltpu.VMEM((CHUNK, DEGW), jnp.float32),
            pltpu.VMEM_SHARED((NPAD, DEGW), jnp.float32),
        ],
    )
    return f(d1, d2, zeros8, ones8)


def _agg_body(xws1, xws2, s1, d1, s2, d2, out1, out2,
              sidx_v, didx_v, rows_v, acc, sem):
    c = lax.axis_index("c")
    s = lax.axis_index("s")

    def work(xws_ref, s_ref, d_ref, out_ref):
        pltpu.sync_copy(xws_ref.at[pl.ds(s * RPT, RPT)],
                        acc.at[pl.ds(s * RPT, RPT)])
        plsc.subcore_barrier()

        def chunk(j, carry):
            base = s * EPT + j * CHUNK
            pltpu.sync_copy(s_ref.at[pl.ds(base, CHUNK)], sidx_v)
            pltpu.sync_copy(d_ref.at[pl.ds(base, CHUNK)], didx_v)
            pltpu.async_copy(xws_ref.at[sidx_v], rows_v, sem).wait()
            pltpu.sync_copy(rows_v, acc.at[didx_v], add=True)
            return carry

        lax.fori_loop(0, NCHUNK, chunk, 0)
        plsc.subcore_barrier()
        pltpu.sync_copy(acc.at[pl.ds(s * RPT, RPT)],
                        out_ref.at[pl.ds(s * RPT, RPT)])

    @pl.when(c == 0)
    def _():
        work(xws1, s1, d1, out1)

    @pl.when(c == 1)
    def _():
        work(xws2, s2, d2, out2)


@functools.cache
def _sc_agg(C):
    return pl.kernel(
        _agg_body,
        out_type=(jax.ShapeDtypeStruct((NPAD, C), jnp.float32),
                  jax.ShapeDtypeStruct((NPAD, C), jnp.float32)),
        mesh=_MESH,
        scratch_types=[
            pltpu.VMEM((CHUNK,), jnp.int32),
            pltpu.VMEM((CHUNK,), jnp.int32),
            pltpu.VMEM((CHUNK, C), jnp.float32),
            pltpu.VMEM_SHARED((NPAD, C), jnp.float32),
            pltpu.SemaphoreType.DMA,
        ],
    )


BLK = 16


def _agg_blk_body(xws1, xws2, s1, d1, s2, d2, out1, out2,
                  sblk, dblk, rows_v, acc, sem):
    c = lax.axis_index("c")
    s = lax.axis_index("s")

    def work(xws_ref, s_ref, d_ref, out_ref):
        pltpu.sync_copy(xws_ref.at[pl.ds(s * RPT, RPT)],
                        acc.at[pl.ds(s * RPT, RPT)])
        plsc.subcore_barrier()

        def blk(m, carry):
            row0 = s * NCHUNK + m * BLK
            pltpu.sync_copy(s_ref.at[pl.ds(row0, BLK)], sblk)
            pltpu.sync_copy(d_ref.at[pl.ds(row0, BLK)], dblk)
            for j in range(BLK):
                pltpu.async_copy(xws_ref.at[sblk.at[j]], rows_v, sem).wait()
                pltpu.sync_copy(rows_v, acc.at[dblk.at[j]], add=True)
            return carry

        lax.fori_loop(0, NCHUNK // BLK, blk, 0)
        plsc.subcore_barrier()
        pltpu.sync_copy(acc.at[pl.ds(s * RPT, RPT)],
                        out_ref.at[pl.ds(s * RPT, RPT)])

    @pl.when(c == 0)
    def _():
        work(xws1, s1, d1, out1)

    @pl.when(c == 1)
    def _():
        work(xws2, s2, d2, out2)


@functools.cache
def _sc_agg_blk(C):
    return pl.kernel(
        _agg_blk_body,
        out_type=(jax.ShapeDtypeStruct((NPAD, C), jnp.float32),
                  jax.ShapeDtypeStruct((NPAD, C), jnp.float32)),
        mesh=_MESH,
        scratch_types=[
            pltpu.VMEM((BLK, CHUNK), jnp.int32),
            pltpu.VMEM((BLK, CHUNK), jnp.int32),
            pltpu.VMEM((CHUNK, C), jnp.float32),
            pltpu.VMEM_SHARED((NPAD, C), jnp.float32),
            pltpu.SemaphoreType.DMA,
        ],
    )


BR = 1024


def _mm1_body(x_ref, w_ref, deg_ref, xws_ref, dinv_ref):
    dinv = lax.rsqrt(deg_ref[...] + 1.0)
    xw = jnp.dot(x_ref[...], w_ref[...], preferred_element_type=jnp.float32)
    xws_ref[...] = xw * dinv
    dinv_ref[...] = dinv


def _tc_layer1(x, w, degp):
    cin, cout = w.shape
    return pl.pallas_call(
        _mm1_body,
        grid=(NPAD // BR,),
        in_specs=[
            pl.BlockSpec((BR, cin), lambda i: (i, 0)),
            pl.BlockSpec((cin, cout), lambda i: (0, 0)),
            pl.BlockSpec((BR, 1), lambda i: (i, 0)),
        ],
        out_specs=[
            pl.BlockSpec((BR, cout), lambda i: (i, 0)),
            pl.BlockSpec((BR, 1), lambda i: (i, 0)),
        ],
        out_shape=[
            jax.ShapeDtypeStruct((NPAD, cout), jnp.float32),
            jax.ShapeDtypeStruct((NPAD, 1), jnp.float32),
        ],
    )(x, w, degp)


def _mid_body(s_ref, w_ref, dinv_ref, b_ref, out_ref):
    dinv = dinv_ref[...]
    h = jnp.maximum(s_ref[...] * dinv + b_ref[...], 0.0)
    out_ref[...] = jnp.dot(h, w_ref[...],
                           preferred_element_type=jnp.float32) * dinv


def _tc_mid(sagg, w, dinv, b_row):
    cin, cout = w.shape
    return pl.pallas_call(
        _mid_body,
        grid=(NPAD // BR,),
        in_specs=[
            pl.BlockSpec((BR, cin), lambda i: (i, 0)),
            pl.BlockSpec((cin, cout), lambda i: (0, 0)),
            pl.BlockSpec((BR, 1), lambda i: (i, 0)),
            pl.BlockSpec((1, cin), lambda i: (0, 0)),
        ],
        out_specs=pl.BlockSpec((BR, cout), lambda i: (i, 0)),
        out_shape=jax.ShapeDtypeStruct((NPAD, cout), jnp.float32),
    )(sagg, w, dinv, b_row)


def _tail_body(s31, s32, dinv1, dinv2, b3r, attw, w2, vt, ntnbT,
               fcw, fcbr, outw, outbr, out_ref):
    def pool(s_ref, dinv_ref):
        h3 = s_ref[...] * dinv_ref[...] + b3r[...]
        ga = jnp.dot(h3, attw[...], preferred_element_type=jnp.float32)
        gc = jnp.mean(ga, axis=0, keepdims=True)
        tg = jnp.tanh(gc)
        sig = jax.nn.sigmoid(jnp.sum(h3 * tg, axis=1, keepdims=True))
        return jnp.sum(h3 * sig, axis=0, keepdims=True)

    g1 = pool(s31, dinv1)
    g2 = pool(s32, dinv2)
    tmp = jnp.dot(g1, w2[...], preferred_element_type=jnp.float32)
    parts = [
        jnp.sum(tmp[:, t * 32:(t + 1) * 32] * g2, axis=1, keepdims=True)
        for t in range(16)
    ]
    scoring = jnp.concatenate(parts, axis=1)
    comb = jnp.concatenate([g1, g2], axis=1)
    block = jnp.dot(comb, vt[...], preferred_element_type=jnp.float32)
    scores = jnp.maximum(scoring + block + ntnbT[...], 0.0)
    h = jnp.maximum(
        jnp.dot(scores, fcw[...], preferred_element_type=jnp.float32)
        + fcbr[...], 0.0)
    out_ref[...] = jax.nn.sigmoid(
        jnp.dot(h, outw[...], preferred_element_type=jnp.float32)
        + outbr[...])


def _tc_tail(s31, s32, dinv1, dinv2, b3r, attw, w2, vt, ntnbT,
             fcw, fcbr, outw, outbr):
    return pl.pallas_call(
        _tail_body,
        out_shape=jax.ShapeDtypeStruct((1, 1), jnp.float32),
    )(s31, s32, dinv1, dinv2, b3r, attw, w2, vt, ntnbT,
      fcw, fcbr, outw, outbr)


def kernel(features_1, features_2, edges_1, edges_2, W1, b1, W2, b2, W3, b3,
           att_W, ntn_W, ntn_V, ntn_b, fc_W, fc_b, out_W, out_b):
    e1 = edges_1.astype(jnp.int32)
    e2 = edges_2.astype(jnp.int32)
    pad = jnp.full((EPAD - E,), NPAD - 1, jnp.int32)
    src1 = jnp.concatenate([e1[0], pad])
    dst1 = jnp.concatenate([e1[1], pad])
    src2 = jnp.concatenate([e2[0], pad])
    dst2 = jnp.concatenate([e2[1], pad])
    shape2d = (EPAD // CHUNK, CHUNK)
    src1b = src1.reshape(shape2d)
    dst1b = dst1.reshape(shape2d)
    src2b = src2.reshape(shape2d)
    dst2b = dst2.reshape(shape2d)

    zeros8 = jnp.zeros((NPAD, DEGW), jnp.float32)
    ones8 = jnp.ones((CHUNK, DEGW), jnp.float32)
    deg1, deg2 = _sc_degree(dst1, dst2, zeros8, ones8)
    deg1 = deg1[:, :1]
    deg2 = deg2[:, :1]

    x1 = jnp.pad(features_1, ((0, NPAD - N), (0, 0)))
    x2 = jnp.pad(features_2, ((0, NPAD - N), (0, 0)))
    xws1, dinv1 = _tc_layer1(x1, W1, deg1)
    xws2, dinv2 = _tc_layer1(x2, W1, deg2)

    W2p = jnp.pad(W2, ((0, 0), (0, 128 - W2.shape[1])))
    W3p = jnp.pad(W3, ((0, 128 - W3.shape[0]), (0, 128 - W3.shape[1])))
    b2p = jnp.pad(b2, (0, 128 - b2.shape[0]))

    s11, s12 = _sc_agg_blk(128)(xws1, xws2, src1b, dst1b, src2b, dst2b)
    b1r = b1.reshape(1, -1)
    xws1 = _tc_mid(s11, W2p, dinv1, b1r)
    xws2 = _tc_mid(s12, W2p, dinv2, b1r)

    s21, s22 = _sc_agg_blk(128)(xws1, xws2, src1b, dst1b, src2b, dst2b)
    b2r = b2p.reshape(1, -1)
    xws1 = _tc_mid(s21, W3p, dinv1, b2r)
    xws2 = _tc_mid(s22, W3p, dinv2, b2r)

    s31, s32 = _sc_agg_blk(128)(xws1, xws2, src1b, dst1b, src2b, dst2b)

    w2t = ntn_W.transpose(0, 2, 1).reshape(32, 512)
    return _tc_tail(
        s31[:N, :32], s32[:N, :32], dinv1[:N], dinv2[:N],
        b3.reshape(1, -1), att_W, w2t, ntn_V.T, ntn_b.reshape(1, -1),
        fc_W, fc_b.reshape(1, -1), out_W, out_b.reshape(1, -1))


# speedup vs baseline: 8.0978x; 1.0776x over previous
import functools

import jax
import jax.numpy as jnp
from jax import lax
from jax.experimental import pallas as pl
from jax.experimental.pallas import tpu as pltpu
from jax.experimental.pallas import tpu_sc as plsc

N = 10000
E = 320000
NS = 16
NPAD = 10240
RPT = NPAD // NS
CHUNK = 128
EPT = 160 * CHUNK
EPAD = NS * EPT
NCHUNK = EPT // CHUNK
DEGW = 8

_MESH = plsc.VectorSubcoreMesh(core_axis_name="c", subcore_axis_name="s")


def _deg_body(d1, d2, zeros8, ones8, deg1, deg2, didx_v, ones_v, acc):
    c = lax.axis_index("c")
    s = lax.axis_index("s")

    def work(d_ref, deg_ref):
        pltpu.sync_copy(zeros8.at[pl.ds(s * RPT, RPT)],
                        acc.at[pl.ds(s * RPT, RPT)])
        pltpu.sync_copy(ones8, ones_v)
        plsc.subcore_barrier()

        def blk(m, carry):
            row0 = s * NCHUNK + m * BLK
            pltpu.sync_copy(d_ref.at[pl.ds(row0, BLK)], didx_v)
            for j in range(BLK):
                pltpu.sync_copy(ones_v, acc.at[didx_v.at[j]], add=True)
            return carry

        lax.fori_loop(0, NCHUNK // BLK, blk, 0)
        plsc.subcore_barrier()
        pltpu.sync_copy(acc.at[pl.ds(s * RPT, RPT)],
                        deg_ref.at[pl.ds(s * RPT, RPT)])

    @pl.when(c == 0)
    def _():
        work(d1, deg1)

    @pl.when(c == 1)
    def _():
        work(d2, deg2)


def _sc_degree(d1, d2, zeros8, ones8):
    f = pl.kernel(
        _deg_body,
        out_type=(jax.ShapeDtypeStruct((NPAD, DEGW), jnp.float32),
                  jax.ShapeDtypeStruct((NPAD, DEGW), jnp.float32)),
        mesh=_MESH,
        scratch_types=[
            pltpu.VMEM((BLK, CHUNK), jnp.int32),
            pltpu.VMEM((CHUNK, DEGW), jnp.float32),
            pltpu.VMEM_SHARED((NPAD, DEGW), jnp.float32),
        ],
    )
    return f(d1, d2, zeros8, ones8)


def _agg_body(xws1, xws2, s1, d1, s2, d2, out1, out2,
              sidx_v, didx_v, rows_v, acc, sem):
    c = lax.axis_index("c")
    s = lax.axis_index("s")

    def work(xws_ref, s_ref, d_ref, out_ref):
        pltpu.sync_copy(xws_ref.at[pl.ds(s * RPT, RPT)],
                        acc.at[pl.ds(s * RPT, RPT)])
        plsc.subcore_barrier()

        def chunk(j, carry):
            base = s * EPT + j * CHUNK
            pltpu.sync_copy(s_ref.at[pl.ds(base, CHUNK)], sidx_v)
            pltpu.sync_copy(d_ref.at[pl.ds(base, CHUNK)], didx_v)
            pltpu.async_copy(xws_ref.at[sidx_v], rows_v, sem).wait()
            pltpu.sync_copy(rows_v, acc.at[didx_v], add=True)
            return carry

        lax.fori_loop(0, NCHUNK, chunk, 0)
        plsc.subcore_barrier()
        pltpu.sync_copy(acc.at[pl.ds(s * RPT, RPT)],
                        out_ref.at[pl.ds(s * RPT, RPT)])

    @pl.when(c == 0)
    def _():
        work(xws1, s1, d1, out1)

    @pl.when(c == 1)
    def _():
        work(xws2, s2, d2, out2)


@functools.cache
def _sc_agg(C):
    return pl.kernel(
        _agg_body,
        out_type=(jax.ShapeDtypeStruct((NPAD, C), jnp.float32),
                  jax.ShapeDtypeStruct((NPAD, C), jnp.float32)),
        mesh=_MESH,
        scratch_types=[
            pltpu.VMEM((CHUNK,), jnp.int32),
            pltpu.VMEM((CHUNK,), jnp.int32),
            pltpu.VMEM((CHUNK, C), jnp.float32),
            pltpu.VMEM_SHARED((NPAD, C), jnp.float32),
            pltpu.SemaphoreType.DMA,
        ],
    )


BLK = 16


def _agg_blk_body(xws1, xws2, s1, d1, s2, d2, out1, out2,
                  sblk, dblk, rows_a, rows_b, acc, sem, sem_s):
    c = lax.axis_index("c")
    s = lax.axis_index("s")

    def work(xws_ref, s_ref, d_ref, out_ref):
        pltpu.sync_copy(xws_ref.at[pl.ds(s * RPT, RPT)],
                        acc.at[pl.ds(s * RPT, RPT)])
        plsc.subcore_barrier()

        def blk(m, carry):
            row0 = s * NCHUNK + m * BLK
            pltpu.sync_copy(s_ref.at[pl.ds(row0, BLK)], sblk)
            pltpu.sync_copy(d_ref.at[pl.ds(row0, BLK)], dblk)
            for j in range(BLK):
                pltpu.async_copy(xws_ref.at[sblk.at[j]], rows_a,
                                 sem).wait()
                pltpu.sync_copy(rows_a, acc.at[dblk.at[j]], add=True)
            return carry

        lax.fori_loop(0, NCHUNK // BLK, blk, 0)
        plsc.subcore_barrier()
        pltpu.sync_copy(acc.at[pl.ds(s * RPT, RPT)],
                        out_ref.at[pl.ds(s * RPT, RPT)])

    @pl.when(c == 0)
    def _():
        work(xws1, s1, d1, out1)

    @pl.when(c == 1)
    def _():
        work(xws2, s2, d2, out2)


@functools.cache
def _sc_agg_blk(C):
    return pl.kernel(
        _agg_blk_body,
        out_type=(jax.ShapeDtypeStruct((NPAD, C), jnp.float32),
                  jax.ShapeDtypeStruct((NPAD, C), jnp.float32)),
        mesh=_MESH,
        scratch_types=[
            pltpu.VMEM((BLK, CHUNK), jnp.int32),
            pltpu.VMEM((BLK, CHUNK), jnp.int32),
            pltpu.VMEM((CHUNK, C), jnp.float32),
            pltpu.VMEM((CHUNK, C), jnp.float32),
            pltpu.VMEM_SHARED((NPAD, C), jnp.float32),
            pltpu.SemaphoreType.DMA,
            pltpu.SemaphoreType.DMA,
        ],
    )


BR = 1024


def _mm1_body(x_ref, w_ref, xw_ref):
    xw_ref[...] = jnp.dot(x_ref[...], w_ref[...],
                          preferred_element_type=jnp.float32)


def _tc_mm1(x, w):
    cin, cout = w.shape
    return pl.pallas_call(
        _mm1_body,
        grid=(NPAD // BR,),
        in_specs=[
            pl.BlockSpec((BR, cin), lambda i: (i, 0)),
            pl.BlockSpec((cin, cout), lambda i: (0, 0)),
        ],
        out_specs=pl.BlockSpec((BR, cout), lambda i: (i, 0)),
        out_shape=jax.ShapeDtypeStruct((NPAD, cout), jnp.float32),
    )(x, w)


def _scale_body(xw_ref, deg_ref, xws_ref, dinv_ref):
    dinv = lax.rsqrt(deg_ref[...] + 1.0)
    xws_ref[...] = xw_ref[...] * dinv
    dinv_ref[...] = dinv


def _tc_scale(xw, degp):
    cout = xw.shape[1]
    return pl.pallas_call(
        _scale_body,
        grid=(NPAD // BR,),
        in_specs=[
            pl.BlockSpec((BR, cout), lambda i: (i, 0)),
            pl.BlockSpec((BR, 1), lambda i: (i, 0)),
        ],
        out_specs=[
            pl.BlockSpec((BR, cout), lambda i: (i, 0)),
            pl.BlockSpec((BR, 1), lambda i: (i, 0)),
        ],
        out_shape=[
            jax.ShapeDtypeStruct((NPAD, cout), jnp.float32),
            jax.ShapeDtypeStruct((NPAD, 1), jnp.float32),
        ],
    )(xw, degp)


def _mid_body(s_ref, w_ref, dinv_ref, b_ref, out_ref):
    dinv = dinv_ref[...]
    h = jnp.maximum(s_ref[...] * dinv + b_ref[...], 0.0)
    out_ref[...] = jnp.dot(h, w_ref[...],
                           preferred_element_type=jnp.float32) * dinv


def _tc_mid(sagg, w, dinv, b_row):
    cin, cout = w.shape
    return pl.pallas_call(
        _mid_body,
        grid=(NPAD // BR,),
        in_specs=[
            pl.BlockSpec((BR, cin), lambda i: (i, 0)),
            pl.BlockSpec((cin, cout), lambda i: (0, 0)),
            pl.BlockSpec((BR, 1), lambda i: (i, 0)),
            pl.BlockSpec((1, cin), lambda i: (0, 0)),
        ],
        out_specs=pl.BlockSpec((BR, cout), lambda i: (i, 0)),
        out_shape=jax.ShapeDtypeStruct((NPAD, cout), jnp.float32),
    )(sagg, w, dinv, b_row)


def _tail_body(s31, s32, dinv1, dinv2, b3r, attw, w2, vt, ntnbT,
               fcw, fcbr, outw, outbr, out_ref):
    def pool(s_ref, dinv_ref):
        h3 = s_ref[...] * dinv_ref[...] + b3r[...]
        ga = jnp.dot(h3, attw[...], preferred_element_type=jnp.float32)
        gc = jnp.mean(ga, axis=0, keepdims=True)
        tg = jnp.tanh(gc)
        sig = jax.nn.sigmoid(jnp.sum(h3 * tg, axis=1, keepdims=True))
        return jnp.sum(h3 * sig, axis=0, keepdims=True)

    g1 = pool(s31, dinv1)
    g2 = pool(s32, dinv2)
    tmp = jnp.dot(g1, w2[...], preferred_element_type=jnp.float32)
    parts = [
        jnp.sum(tmp[:, t * 32:(t + 1) * 32] * g2, axis=1, keepdims=True)
        for t in range(16)
    ]
    scoring = jnp.concatenate(parts, axis=1)
    comb = jnp.concatenate([g1, g2], axis=1)
    block = jnp.dot(comb, vt[...], preferred_element_type=jnp.float32)
    scores = jnp.maximum(scoring + block + ntnbT[...], 0.0)
    h = jnp.maximum(
        jnp.dot(scores, fcw[...], preferred_element_type=jnp.float32)
        + fcbr[...], 0.0)
    out_ref[...] = jax.nn.sigmoid(
        jnp.dot(h, outw[...], preferred_element_type=jnp.float32)
        + outbr[...])


def _tc_tail(s31, s32, dinv1, dinv2, b3r, attw, w2, vt, ntnbT,
             fcw, fcbr, outw, outbr):
    return pl.pallas_call(
        _tail_body,
        out_shape=jax.ShapeDtypeStruct((1, 1), jnp.float32),
    )(s31, s32, dinv1, dinv2, b3r, attw, w2, vt, ntnbT,
      fcw, fcbr, outw, outbr)


def kernel(features_1, features_2, edges_1, edges_2, W1, b1, W2, b2, W3, b3,
           att_W, ntn_W, ntn_V, ntn_b, fc_W, fc_b, out_W, out_b):
    e1 = edges_1.astype(jnp.int32)
    e2 = edges_2.astype(jnp.int32)
    pad = jnp.full((EPAD - E,), NPAD - 1, jnp.int32)
    src1 = jnp.concatenate([e1[0], pad])
    dst1 = jnp.concatenate([e1[1], pad])
    src2 = jnp.concatenate([e2[0], pad])
    dst2 = jnp.concatenate([e2[1], pad])
    shape2d = (EPAD // CHUNK, CHUNK)
    src1b = src1.reshape(shape2d)
    dst1b = dst1.reshape(shape2d)
    src2b = src2.reshape(shape2d)
    dst2b = dst2.reshape(shape2d)

    zeros8 = jnp.zeros((NPAD, DEGW), jnp.float32)
    ones8 = jnp.ones((CHUNK, DEGW), jnp.float32)
    deg1, deg2 = _sc_degree(dst1b, dst2b, zeros8, ones8)
    deg1 = deg1[:, :1]
    deg2 = deg2[:, :1]

    x1 = jnp.pad(features_1, ((0, NPAD - N), (0, 0)))
    x2 = jnp.pad(features_2, ((0, NPAD - N), (0, 0)))
    xw1 = _tc_mm1(x1, W1)
    xw2 = _tc_mm1(x2, W1)
    xws1, dinv1 = _tc_scale(xw1, deg1)
    xws2, dinv2 = _tc_scale(xw2, deg2)

    W2p = jnp.pad(W2, ((0, 0), (0, 128 - W2.shape[1])))
    W3p = jnp.pad(W3, ((0, 128 - W3.shape[0]), (0, 128 - W3.shape[1])))
    b2p = jnp.pad(b2, (0, 128 - b2.shape[0]))

    s11, s12 = _sc_agg_blk(128)(xws1, xws2, src1b, dst1b, src2b, dst2b)
    b1r = b1.reshape(1, -1)
    xws1 = _tc_mid(s11, W2p, dinv1, b1r)
    xws2 = _tc_mid(s12, W2p, dinv2, b1r)

    s21, s22 = _sc_agg_blk(128)(xws1, xws2, src1b, dst1b, src2b, dst2b)
    b2r = b2p.reshape(1, -1)
    xws1 = _tc_mid(s21, W3p, dinv1, b2r)
    xws2 = _tc_mid(s22, W3p, dinv2, b2r)

    s31, s32 = _sc_agg_blk(128)(xws1, xws2, src1b, dst1b, src2b, dst2b)

    w2t = ntn_W.transpose(0, 2, 1).reshape(32, 512)
    return _tc_tail(
        s31[:N, :32], s32[:N, :32], dinv1[:N], dinv2[:N],
        b3.reshape(1, -1), att_W, w2t, ntn_V.T, ntn_b.reshape(1, -1),
        fc_W, fc_b.reshape(1, -1), out_W, out_b.reshape(1, -1))
